# Initial kernel scaffold; baseline (speedup 1.0000x reference)
import jax
import jax.numpy as jnp
from jax.experimental import pallas as pl


def kernel(x, edge_index, W1, b1, W2, b2, W3, b3):
    raise NotImplementedError("write your pallas kernel here")


# speedup vs baseline: 7.8369x; 7.8369x over previous
import jax
import jax.numpy as jnp
from jax import lax
from jax.experimental import pallas as pl
from jax.experimental.pallas import tpu as pltpu
from jax.experimental.pallas import tpu_sc as plsc

N, E, D = 10000, 320000, 128
NC, NS = 2, 16
NW = NC * NS
EW = E // NW
CH = 125
NCHUNK = EW // CH
RPT = N // NS
DW = 16
WB = 200
F32 = jnp.float32
_MESH = plsc.VectorSubcoreMesh(core_axis_name="c", subcore_axis_name="s")


def _writeback(c, s, copies):
    nfull = N // WB // NS
    for j in range(nfull):
        copies(j * NS + s)
    rem = N // WB - nfull * NS

    @pl.when(s < rem)
    def _():
        copies(nfull * NS + s)


def _sc_agg_body(h_hbm, src_hbm, dst_hbm, agg_out,
                 src_v, dst_v, rows_v, agg_sh, sem):
    c = lax.axis_index("c")
    s = lax.axis_index("s")
    wid = s * NC + c
    base = s * RPT

    zero16 = jnp.zeros((16,), F32)

    def zfill(i, _):
        for k in range(D // 16):
            rows_v[i, pl.ds(16 * k, 16)] = zero16
        return 0

    lax.fori_loop(0, CH, zfill, 0)

    for i in range(RPT // CH):
        pltpu.sync_copy(rows_v, agg_sh.at[pl.ds(base + i * CH, CH)])

    pltpu.sync_copy(src_hbm.at[wid], src_v)
    pltpu.sync_copy(dst_hbm.at[wid], dst_v)

    plsc.subcore_barrier()

    def chunk(j, _):
        pltpu.async_copy(h_hbm.at[src_v.at[j]], rows_v, sem).wait()
        pltpu.sync_copy(rows_v, agg_sh.at[dst_v.at[j]], add=True)
        return 0

    lax.fori_loop(0, NCHUNK, chunk, 0)

    plsc.subcore_barrier()

    def wb(cid):
        sl = pl.ds(cid * WB, WB)
        pltpu.sync_copy(agg_sh.at[sl], agg_out.at[c, sl])

    _writeback(c, s, wb)


_sc_agg = pl.kernel(
    _sc_agg_body,
    out_type=(jax.ShapeDtypeStruct((NC, N, D), F32),),
    mesh=_MESH,
    scratch_types=[
        pltpu.VMEM((NCHUNK, CH), jnp.int32),
        pltpu.VMEM((NCHUNK, CH), jnp.int32),
        pltpu.VMEM((CH, D), F32),
        pltpu.VMEM_SHARED((N, D), F32),
        pltpu.SemaphoreType.DMA,
    ],
)


def _sc_deg_body(dst_hbm, deg_out, dst_v, ones_v, deg_sh):
    c = lax.axis_index("c")
    s = lax.axis_index("s")
    wid = s * NC + c
    base = s * RPT

    zero16 = jnp.zeros((16,), F32)
    one16 = jnp.full((16,), 1.0, F32)

    def zfill(i, _):
        for k in range(D // 16):
            ones_v[i, pl.ds(16 * k, 16)] = zero16
        return 0

    lax.fori_loop(0, CH, zfill, 0)

    for i in range(RPT // CH):
        pltpu.sync_copy(ones_v, deg_sh.at[pl.ds(base + i * CH, CH)])

    def ofill(i, _):
        for k in range(D // 16):
            ones_v[i, pl.ds(16 * k, 16)] = one16
        return 0

    lax.fori_loop(0, CH, ofill, 0)

    pltpu.sync_copy(dst_hbm.at[wid], dst_v)

    plsc.subcore_barrier()

    def chunk(j, _):
        pltpu.sync_copy(ones_v, deg_sh.at[dst_v.at[j]], add=True)
        return 0

    lax.fori_loop(0, NCHUNK, chunk, 0)

    plsc.subcore_barrier()

    def wb(cid):
        sl = pl.ds(cid * WB, WB)
        pltpu.sync_copy(deg_sh.at[sl], deg_out.at[c, sl])

    _writeback(c, s, wb)


_sc_deg = pl.kernel(
    _sc_deg_body,
    out_type=(jax.ShapeDtypeStruct((NC, N, D), F32),),
    mesh=_MESH,
    scratch_types=[
        pltpu.VMEM((NCHUNK, CH), jnp.int32),
        pltpu.VMEM((CH, D), F32),
        pltpu.VMEM_SHARED((N, D), F32),
    ],
)


def _tc_layer(h, parts, degp, W, b, mode):
    BN = 1000

    def body(h_ref, p_ref, dg_ref, w_ref, b_ref, o_ref):
        dg = dg_ref[0, :, 0:1] + dg_ref[1, :, 0:1]
        inv = 1.0 / jnp.maximum(dg, 1.0)
        agg = (p_ref[0] + p_ref[1]) * inv
        acc = (
            jnp.dot(h_ref[...], w_ref[:D],
                    preferred_element_type=F32,
                    precision=lax.Precision.HIGHEST)
            + jnp.dot(agg, w_ref[D:],
                      preferred_element_type=F32,
                      precision=lax.Precision.HIGHEST)
            + b_ref[...]
        )
        if mode == "relu":
            o_ref[...] = jnp.maximum(acc, 0.0)
        else:
            m = jnp.max(acc, axis=-1, keepdims=True)
            lse = jnp.log(jnp.sum(jnp.exp(acc - m), axis=-1, keepdims=True)) + m
            o_ref[...] = acc - lse

    return pl.pallas_call(
        body,
        grid=(N // BN,),
        in_specs=[
            pl.BlockSpec((BN, D), lambda i: (i, 0)),
            pl.BlockSpec((NC, BN, D), lambda i: (0, i, 0)),
            pl.BlockSpec((NC, BN, D), lambda i: (0, i, 0)),
            pl.BlockSpec((2 * D, D), lambda i: (0, 0)),
            pl.BlockSpec((1, D), lambda i: (0, 0)),
        ],
        out_specs=pl.BlockSpec((BN, D), lambda i: (i, 0)),
        out_shape=jax.ShapeDtypeStruct((N, D), F32),
    )(h, parts, degp, W, b.reshape(1, D))


def kernel(x, edge_index, W1, b1, W2, b2, W3, b3):
    src = edge_index[0].reshape(NW, NCHUNK, CH)
    dst = edge_index[1].reshape(NW, NCHUNK, CH)
    (degp,) = _sc_deg(dst)
    (agg1,) = _sc_agg(x, src, dst)
    h1 = _tc_layer(x, agg1, degp, W1, b1, "relu")
    (agg2,) = _sc_agg(h1, src, dst)
    h2 = _tc_layer(h1, agg2, degp, W2, b2, "relu")
    (agg3,) = _sc_agg(h2, src, dst)
    return _tc_layer(h2, agg3, degp, W3, b3, "logsoftmax")
